# baseline (speedup 1.0000x reference)
import functools

import jax
import jax.numpy as jnp
from jax import lax
from jax.experimental import pallas as pl
from jax.experimental.pallas import tpu as pltpu
from jax.experimental.pallas import tpu_sc as plsc

MS = 512
B = 128
N = MS * (MS + 1) // 2
RB = 8
GB = 8
NG = B // GB
L = 16
NJ = MS // L

WIN = 2176

_info = plsc.get_sparse_core_info()
_NC, _NS = _info.num_cores, _info.num_subcores
_NW = _NC * _NS


def kernel(inputs):
    @functools.partial(
        pl.kernel,
        mesh=plsc.VectorSubcoreMesh(core_axis_name="c", subcore_axis_name="s"),
        out_type=jax.ShapeDtypeStruct((B, MS, MS), jnp.float32),
        scratch_types=[
            pltpu.VMEM((GB, WIN), jnp.float32),
            pltpu.VMEM((GB, WIN), jnp.float32),
            pltpu.VMEM((GB, WIN), jnp.float32),
            pltpu.VMEM((GB, RB, MS), jnp.float32),
            pltpu.VMEM((GB, RB, MS), jnp.float32),
            pltpu.SemaphoreType.DMA,
            pltpu.SemaphoreType.DMA,
            pltpu.SemaphoreType.DMA,
            pltpu.SemaphoreType.DMA,
            pltpu.SemaphoreType.DMA,
        ],
        compiler_params=pltpu.CompilerParams(needs_layout_passes=False),
    )
    def sc_kernel(
        in_hbm, out_hbm, inA, inB, inC, o0, o1, sA, sB, sC, so0, so1
    ):
        wid = lax.axis_index("s") * _NC + lax.axis_index("c")
        lane = jnp.arange(L, dtype=jnp.int32)
        zero = jnp.zeros((L,), jnp.float32)

        def span(r0):
            off0 = MS * r0 - (r0 * (r0 - 1)) // 2
            sp = lax.min((off0 // 128) * 128, N - WIN)
            return pl.multiple_of(sp, 128)

        r0_0 = wid * RB
        r0_1 = (wid + 32) * RB
        sp_a = span(r0_0)
        sp_b = span(r0_0 + 4)
        sp_c = span(r0_1)

        def in_src(b0, sp):
            return in_hbm.at[pl.ds(b0, GB), pl.ds(sp, WIN)]

        def out_dst(b0, r0):
            return out_hbm.at[
                pl.ds(b0, GB), pl.ds(pl.multiple_of(r0, RB), RB), :
            ]

        def compute(in_ref, sp, r0, rows_lo, nrows, out_ref, skip):
            @plsc.parallel_loop(0, GB * nrows, unroll=4)
            def _(i):
                bb = i // nrows
                rl = rows_lo + i % nrows
                row_idx = jnp.zeros((L,), dtype=jnp.int32) + bb
                r = r0 + rl
                off_r = MS * r - (r * (r - 1)) // 2
                bl = off_r - r - sp
                lr = lane - r
                for j in range(NJ):
                    c0 = j * L
                    if c0 + L <= skip:
                        continue
                    col = lane + (bl + c0)
                    m = lr >= jnp.int32(-c0)
                    v = plsc.load_gather(in_ref, [row_idx, col], mask=m)
                    out_ref[bb, rl, pl.ds(c0, L)] = v

        def per_group(tt, carry):
            b0 = pl.multiple_of(tt * GB, GB)
            b1 = pl.multiple_of(b0 + GB, GB)
            not_last = tt < NG - 1

            pltpu.make_async_copy(in_src(b0, sp_a), inA, sA).wait()
            @pl.when(tt > 0)
            def _():
                pltpu.make_async_copy(o0, out_dst(b0, r0_0), so0).wait()
            compute(inA, sp_a, r0_0, 0, 4, o0, 0)
            @pl.when(not_last)
            def _():
                pltpu.async_copy(in_src(b1, sp_a), inA, sA)

            pltpu.make_async_copy(in_src(b0, sp_b), inB, sB).wait()
            compute(inB, sp_b, r0_0, 4, 4, o0, 0)
            @pl.when(not_last)
            def _():
                pltpu.async_copy(in_src(b1, sp_b), inB, sB)
            pltpu.async_copy(o0, out_dst(b0, r0_0), so0)

            pltpu.make_async_copy(in_src(b0, sp_c), inC, sC).wait()
            @pl.when(tt > 0)
            def _():
                pltpu.make_async_copy(o1, out_dst(b0, r0_1), so1).wait()
            compute(inC, sp_c, r0_1, 0, 8, o1, 256)
            @pl.when(not_last)
            def _():
                pltpu.async_copy(in_src(b1, sp_c), inC, sC)
            pltpu.async_copy(o1, out_dst(b0, r0_1), so1)
            return carry

        @plsc.parallel_loop(0, GB * RB, unroll=2)
        def _(i):
            bb = i // RB
            rl = i % RB
            for j in range(NJ // 2):
                o1[bb, rl, pl.ds(j * L, L)] = zero

        pltpu.async_copy(in_src(0, sp_a), inA, sA)
        pltpu.async_copy(in_src(0, sp_b), inB, sB)
        pltpu.async_copy(in_src(0, sp_c), inC, sC)
        lax.fori_loop(0, NG, per_group, 0)
        pltpu.make_async_copy(o0, out_dst(0, r0_0), so0).wait()
        pltpu.make_async_copy(o1, out_dst(0, r0_1), so1).wait()

    return sc_kernel(inputs)


# speedup vs baseline: 2.0061x; 2.0061x over previous
import functools

import jax
import jax.numpy as jnp
from jax import lax
from jax.experimental import pallas as pl
from jax.experimental.pallas import tpu as pltpu
from jax.experimental.pallas import tpu_sc as plsc

MS = 512
B = 128
N = MS * (MS + 1) // 2
RB = 8
GB = 8
NG = B // GB
L = 16
NJ = MS // L

WIN0 = 4352
WIN1 = 2176

_info = plsc.get_sparse_core_info()
_NC, _NS = _info.num_cores, _info.num_subcores
_NW = _NC * _NS


def kernel(inputs):
    @functools.partial(
        pl.kernel,
        mesh=plsc.VectorSubcoreMesh(core_axis_name="c", subcore_axis_name="s"),
        out_type=jax.ShapeDtypeStruct((B, MS, MS), jnp.float32),
        scratch_types=[
            pltpu.VMEM((GB, WIN0), jnp.float32),
            pltpu.VMEM((GB, WIN1), jnp.float32),
            pltpu.VMEM((GB, RB, MS), jnp.float32),
            pltpu.VMEM((GB, RB, MS), jnp.float32),
            pltpu.SemaphoreType.DMA,
            pltpu.SemaphoreType.DMA,
            pltpu.SemaphoreType.DMA,
            pltpu.SemaphoreType.DMA,
        ],
        compiler_params=pltpu.CompilerParams(needs_layout_passes=False),
    )
    def sc_kernel(in_hbm, out_hbm, in0, in1, o0, o1, s0, s1, so0, so1):
        wid = lax.axis_index("s") * _NC + lax.axis_index("c")
        lane = jnp.arange(L, dtype=jnp.int32)
        zero = jnp.zeros((L,), jnp.float32)

        def span(k, win):
            r0 = k * RB
            off0 = MS * r0 - (r0 * (r0 - 1)) // 2
            sp = lax.min((off0 // 128) * 128, N - win)
            return r0, pl.multiple_of(sp, 128)

        r0_0, sp_0 = span(wid, WIN0)
        r0_1, sp_1 = span(wid + 32, WIN1)

        def in_src(b0, sp, win):
            return in_hbm.at[pl.ds(b0, GB), pl.ds(sp, win)]

        def out_dst(b0, r0):
            return out_hbm.at[
                pl.ds(b0, GB), pl.ds(pl.multiple_of(r0, RB), RB), :
            ]

        def compute(in_ref, sp, r0, out_ref, skip):
            @plsc.parallel_loop(0, GB * RB, unroll=4)
            def _(i):
                bb = i // RB
                rl = i % RB
                row_idx = jnp.zeros((L,), dtype=jnp.int32) + bb
                r = r0 + rl
                off_r = MS * r - (r * (r - 1)) // 2
                bl = off_r - r - sp
                lr = lane - r
                for j in range(NJ):
                    c0 = j * L
                    if c0 + L <= skip:
                        continue
                    col = lane + (bl + c0)
                    m = lr >= jnp.int32(-c0)
                    v = plsc.load_gather(in_ref, [row_idx, col], mask=m)
                    out_ref[bb, rl, pl.ds(c0, L)] = v

        def per_group(tt, carry):
            b0 = pl.multiple_of(tt * GB, GB)
            for p in range(2):
                in_ref, sem = (in0, s0) if p == 0 else (in1, s1)
                sp = sp_0 if p == 0 else sp_1
                r0 = r0_0 if p == 0 else r0_1
                win = WIN0 if p == 0 else WIN1
                out_ref, osem = (o0, so0) if p == 0 else (o1, so1)
                pltpu.make_async_copy(in_src(b0, sp, win), in_ref, sem).wait()
                if p == 0:
                    pltpu.async_copy(in_src(b0, sp_1, WIN1), in1, s1)
                else:
                    @pl.when(tt < NG - 1)
                    def _():
                        pltpu.async_copy(in_src(b0 + GB, sp_0, WIN0), in0, s0)
                @pl.when(tt > 0)
                def _(out_ref=out_ref, osem=osem, r0=r0):
                    pltpu.make_async_copy(
                        out_ref, out_dst(b0, r0), osem
                    ).wait()
                compute(in_ref, sp, r0, out_ref, 256 if p else 0)
                pltpu.async_copy(out_ref, out_dst(b0, r0), osem)
            return carry

        @plsc.parallel_loop(0, GB * RB, unroll=2)
        def _(i):
            bb = i // RB
            rl = i % RB
            for j in range(NJ // 2):
                o1[bb, rl, pl.ds(j * L, L)] = zero

        pltpu.async_copy(in_src(0, sp_0, WIN0), in0, s0)
        lax.fori_loop(0, NG, per_group, 0)
        pltpu.make_async_copy(o0, out_dst(0, r0_0), so0).wait()
        pltpu.make_async_copy(o1, out_dst(0, r0_1), so1).wait()

    return sc_kernel(inputs)
